# baseline (speedup 1.0000x reference)
import functools

import jax
import jax.numpy as jnp
from jax import lax
from jax.experimental import pallas as pl
from jax.experimental.pallas import tpu as pltpu
from jax.experimental.pallas import tpu_sc as plsc

N = 10000
NP = 10240
E = 160000
D = 256
EW = 128
SDW = 16
G = 8
L = 3
CHUNK = 128
NCHUNK = E // CHUNK
NB = 1024
EB = 640
NTILE = 32
ROWS_PER_TILE = NP // 16

_f32 = jnp.float32
_i32 = jnp.int32


def _sc_mesh():
    return plsc.VectorSubcoreMesh(core_axis_name="c", subcore_axis_name="s",
                                  num_cores=2, num_subcores=16)


def _tile_range(s, total):
    per = total // 16
    rem = total - per * 16
    base = s * per + jnp.minimum(s, rem)
    nj = per + (s < rem).astype(_i32)
    return base, nj


def _gather_body(eoff, nchunks, ta, tb, p4, dst, src, outa, outb, outd,
                 idxg, idxo, bufs, ptab, bufd, sem0, sem1, semi0, semi1):
    c = lax.axis_index("c")
    s = lax.axis_index("s")
    base, nj = _tile_range(s, nchunks)

    def run(table, gidx, oidx, out, g_is_dst, dpred):
        @pl.when(dpred)
        def _():
            pltpu.sync_copy(p4, ptab)
        sems = (sem0, sem1)
        isems = (semi0, semi1)

        def idx_load_async(j, slot):
            off = (eoff + base + j) * CHUNK
            pltpu.async_copy(gidx.at[pl.ds(off, CHUNK)], idxg.at[slot],
                             isems[slot])

            @pl.when(dpred)
            def _():
                pltpu.async_copy(oidx.at[pl.ds(off, CHUNK)], idxo.at[slot],
                                 isems[slot])

        def idx_wait(j, slot):
            off = (eoff + base + j) * CHUNK
            pltpu.make_async_copy(gidx.at[pl.ds(off, CHUNK)], idxg.at[slot],
                                  isems[slot]).wait()

            @pl.when(dpred)
            def _():
                pltpu.make_async_copy(oidx.at[pl.ds(off, CHUNK)],
                                      idxo.at[slot], isems[slot]).wait()

        def fire(slot):
            pltpu.async_copy(table.at[idxg.at[slot]], bufs.at[slot], sems[slot])

        def finish(j, slot):
            off = (base + j) * CHUNK

            @pl.when(dpred)
            def _():
                def dc(i, carry2):
                    gv = idxg[slot, pl.ds(i * 16, 16)] * 4
                    ov = idxo[slot, pl.ds(i * 16, 16)] * 4
                    dv, sv = (gv, ov) if g_is_dst else (ov, gv)
                    dx = (plsc.load_gather(ptab, [dv])
                          - plsc.load_gather(ptab, [sv]))
                    dy = (plsc.load_gather(ptab, [dv + 1])
                          - plsc.load_gather(ptab, [sv + 1]))
                    dz = (plsc.load_gather(ptab, [dv + 2])
                          - plsc.load_gather(ptab, [sv + 2]))
                    dist2 = dx * dx + dy * dy + dz * dz
                    bufd[0, pl.ds(i * 16, 16)] = dx
                    bufd[1, pl.ds(i * 16, 16)] = dy
                    bufd[2, pl.ds(i * 16, 16)] = dz
                    bufd[3, pl.ds(i * 16, 16)] = dist2
                    return carry2

                lax.fori_loop(0, CHUNK // 16, dc, None)
                pltpu.sync_copy(bufd, outd.at[:, pl.ds(off, CHUNK)])

            pltpu.make_async_copy(table.at[idxg.at[slot]], bufs.at[slot],
                                  sems[slot]).wait()
            pltpu.sync_copy(bufs.at[slot], out.at[pl.ds(off, CHUNK), :])

        idx_load_async(0, 0)
        idx_wait(0, 0)
        fire(0)

        @pl.when(1 < nj)
        def _():
            idx_load_async(1, 1)

        def step(j, cur, nxt):
            @pl.when(j + 1 < nj)
            def _():
                idx_wait(j + 1, nxt)
                fire(nxt)
            finish(j, cur)

            @pl.when(j + 2 < nj)
            def _():
                idx_load_async(j + 2, cur)

        def lbody(j, carry):
            @pl.when(j % 2 == 0)
            def _():
                step(j, 0, 1)

            @pl.when(j % 2 == 1)
            def _():
                step(j, 1, 0)

            return carry

        lax.fori_loop(0, nj, lbody, None)

    @pl.when(c == 0)
    def _():
        run(ta, dst, src, outa, True, s < 8)

    @pl.when(c == 1)
    def _():
        run(tb, src, dst, outb, False, s >= 8)


PACKW = D // 2


@functools.cache
def _gather_call(half):
    e2 = E // 2
    return pl.kernel(
        functools.partial(_gather_body, half * (NCHUNK // 2), NCHUNK // 2),
        out_type=[jax.ShapeDtypeStruct((e2, PACKW), _i32),
                  jax.ShapeDtypeStruct((e2, PACKW), _i32),
                  jax.ShapeDtypeStruct((4, e2), _f32)],
        mesh=_sc_mesh(),
        compiler_params=pltpu.CompilerParams(needs_layout_passes=False),
        scratch_types=[
            pltpu.VMEM((2, CHUNK), _i32),
            pltpu.VMEM((2, CHUNK), _i32),
            pltpu.VMEM((2, CHUNK, PACKW), _i32),
            pltpu.VMEM((NP * 4,), _f32),
            pltpu.VMEM((4, CHUNK), _f32),
            pltpu.SemaphoreType.DMA,
            pltpu.SemaphoreType.DMA,
            pltpu.SemaphoreType.DMA,
            pltpu.SemaphoreType.DMA,
        ],
    )


def _zero_acc(s, zhbm, acc):
    rows = pl.ds(s * ROWS_PER_TILE, ROWS_PER_TILE)
    pltpu.sync_copy(zhbm.at[rows, :], acc.at[rows, :])


def _scatter_loop(eoff, e, dst, base, nj, idx2, buf2, acc, semi, semb):
    sems = (semi, semb)

    def load(j, slot):
        off = (base + j) * CHUNK
        goff = off + eoff * CHUNK
        pltpu.async_copy(dst.at[pl.ds(goff, CHUNK)], idx2.at[slot], sems[slot])
        pltpu.async_copy(e.at[pl.ds(off, CHUNK), :], buf2.at[slot], sems[slot])

    def finish(j, slot):
        off = (base + j) * CHUNK
        goff = off + eoff * CHUNK
        pltpu.make_async_copy(dst.at[pl.ds(goff, CHUNK)], idx2.at[slot],
                              sems[slot]).wait()
        pltpu.make_async_copy(e.at[pl.ds(off, CHUNK), :], buf2.at[slot],
                              sems[slot]).wait()
        pltpu.sync_copy(buf2.at[slot], acc.at[idx2.at[slot]], add=True)

    load(0, 0)

    def lbody(j, carry):
        @pl.when(j % 2 == 0)
        def _():
            @pl.when(j + 1 < nj)
            def _():
                load(j + 1, 1)
            finish(j, 0)

        @pl.when(j % 2 == 1)
        def _():
            @pl.when(j + 1 < nj)
            def _():
                load(j + 1, 0)
            finish(j, 1)

        return carry

    lax.fori_loop(0, nj, lbody, None)


def _scatter_body(eoff, nchunks, e1, e2, e3, dst, zhbm,
                  agg_a, agg_b, agg3a, agg3b, idx2, buf2, acc, semi, semb):
    c = lax.axis_index("c")
    s = lax.axis_index("s")
    rows = pl.ds(s * ROWS_PER_TILE, ROWS_PER_TILE)
    _zero_acc(s, zhbm, acc)
    plsc.subcore_barrier()

    base, nj = _tile_range(s, nchunks)

    @pl.when(c == 0)
    def _():
        _scatter_loop(eoff, e1, dst, base, nj, idx2, buf2, acc, semi, semb)

    @pl.when(c == 1)
    def _():
        _scatter_loop(eoff, e2, dst, base, nj, idx2, buf2, acc, semi, semb)

    plsc.subcore_barrier()

    @pl.when(c == 0)
    def _():
        pltpu.sync_copy(acc.at[rows, :], agg_a.at[rows, :])

    @pl.when(c == 1)
    def _():
        pltpu.sync_copy(acc.at[rows, :], agg_b.at[rows, :])

    _zero_acc(s, zhbm, acc)
    plsc.subcore_barrier()

    nc0 = nchunks // 2

    @pl.when(c == 0)
    def _():
        base2, nj2 = _tile_range(s, nc0)
        _scatter_loop(eoff, e3, dst, base2, nj2, idx2, buf2, acc, semi, semb)

    @pl.when(c == 1)
    def _():
        base2, nj2 = _tile_range(s, nchunks - nc0)
        _scatter_loop(eoff, e3, dst, nc0 + base2, nj2, idx2, buf2, acc,
                      semi, semb)

    plsc.subcore_barrier()

    @pl.when(c == 0)
    def _():
        pltpu.sync_copy(acc.at[rows, :], agg3a.at[rows, :])

    @pl.when(c == 1)
    def _():
        pltpu.sync_copy(acc.at[rows, :], agg3b.at[rows, :])


@functools.cache
def _scatter_call(half):
    return pl.kernel(
        functools.partial(_scatter_body, half * (NCHUNK // 2), NCHUNK // 2),
        out_type=[jax.ShapeDtypeStruct((NP, EW), _f32)] * 4,
        mesh=_sc_mesh(),
        scratch_types=[
            pltpu.VMEM((2, CHUNK), _i32),
            pltpu.VMEM((2, CHUNK, EW), _f32),
            pltpu.VMEM_SHARED((NP, EW), _f32),
            pltpu.SemaphoreType.DMA,
            pltpu.SemaphoreType.DMA,
        ],
    )


def _input_body(x_r, p_r, wi_r, bi_r, w1a_r, w1b_r, b1_r, h_o, ta_o, tb_o, p4_o):
    h = x_r[...] @ wi_r[...] + bi_r[...]
    h_o[...] = h
    ta_o[...] = h @ w1a_r[...] + b1_r[...]
    tb_o[...] = h @ w1b_r[...]
    p4_o[...] = p_r[...]


def _full(shape):
    return pl.BlockSpec(shape, lambda i: (0, 0))


_input_call = pl.pallas_call(
    _input_body,
    grid=(NP // NB,),
    in_specs=[
        pl.BlockSpec((NB, D), lambda i: (i, 0)),
        pl.BlockSpec((NB, 4), lambda i: (i, 0)),
        _full((D, D)),
        _full((1, D)),
        _full((D, D)),
        _full((D, D)),
        _full((1, D)),
    ],
    out_specs=[
        pl.BlockSpec((NB, D), lambda i: (i, 0)),
        pl.BlockSpec((NB, D), lambda i: (i, 0)),
        pl.BlockSpec((NB, D), lambda i: (i, 0)),
        pl.BlockSpec((NB, 4), lambda i: (i, 0)),
    ],
    out_shape=[
        jax.ShapeDtypeStruct((NP, D), _f32),
        jax.ShapeDtypeStruct((NP, D), _f32),
        jax.ShapeDtypeStruct((NP, D), _f32),
        jax.ShapeDtypeStruct((NP, 4), _f32),
    ],
)


def _edge_body(ai_r, bj_r, sd_r, w2_r, b2_r, w1d_r, posw_r, g_r, bb_r,
               e1_o, e2_o, e3_o):
    sd = jnp.transpose(sd_r[...])
    dcol = sd[:, 0:3]
    dist2 = sd[:, 3:4]
    m = jnp.maximum(ai_r[...] + bj_r[...] + dist2 * w1d_r[...], 0.0)
    m = jnp.maximum(m @ w2_r[...] + b2_r[...], 0.0)
    mu = jnp.mean(m, axis=1, keepdims=True)
    var = jnp.mean(jnp.square(m - mu), axis=1, keepdims=True)
    m = (m - mu) * lax.rsqrt(var + 1e-5) * g_r[...] + bb_r[...]
    wgt = jnp.tanh(jnp.sum(m * posw_r[...], axis=1, keepdims=True))
    dw = dcol * wgt
    ones = jnp.ones((EB, 1), _f32)
    zpad = jnp.zeros((EB, EW - 4), _f32)
    e1_o[...] = m[:, 0:EW]
    e2_o[...] = m[:, EW:D]
    e3_o[...] = jnp.concatenate([dw, ones, zpad], axis=1)


E2 = E // 2

_edge_call = pl.pallas_call(
    _edge_body,
    grid=(E2 // EB,),
    in_specs=[
        pl.BlockSpec((EB, D), lambda i: (i, 0)),
        pl.BlockSpec((EB, D), lambda i: (i, 0)),
        pl.BlockSpec((4, EB), lambda i: (0, i)),
        _full((D, D)),
        _full((1, D)),
        _full((1, D)),
        _full((1, D)),
        _full((1, D)),
        _full((1, D)),
    ],
    out_specs=[
        pl.BlockSpec((EB, EW), lambda i: (i, 0)),
        pl.BlockSpec((EB, EW), lambda i: (i, 0)),
        pl.BlockSpec((EB, EW), lambda i: (i, 0)),
    ],
    out_shape=[
        jax.ShapeDtypeStruct((E2, EW), _f32),
        jax.ShapeDtypeStruct((E2, EW), _f32),
        jax.ShapeDtypeStruct((E2, EW), _f32),
    ],
)


def _make_node_call(has_next):
    def body(*refs):
        if has_next:
            (h_r, p_r, aa0_r, ab0_r, aa1_r, ab1_r, a3a0_r, a3b0_r, a3a1_r,
             a3b1_r, wua_r, wub_r, ub_r, w1a_r, w1b_r, b1_r,
             h_o, p_o, ta_o, tb_o) = refs
        else:
            (h_r, p_r, aa0_r, ab0_r, aa1_r, ab1_r, a3a0_r, a3b0_r, a3a1_r,
             a3b1_r, wua_r, wub_r, ub_r, h_o, p_o) = refs
        a3 = a3a0_r[...] + a3b0_r[...] + a3a1_r[...] + a3b1_r[...]
        deg = jnp.maximum(a3[:, 3:4], 1.0)
        magg = jnp.concatenate([aa0_r[...] + aa1_r[...],
                                ab0_r[...] + ab1_r[...]], axis=1) / deg
        hv = h_r[...]
        hu = jnp.maximum(hv @ wua_r[...] + magg @ wub_r[...] + ub_r[...], 0.0)
        hn = hv + hu
        p3 = p_r[...][:, 0:3] + a3[:, 0:3] / deg
        h_o[...] = hn
        p_o[...] = jnp.concatenate([p3, jnp.zeros((NB, 1), _f32)], axis=1)
        if has_next:
            ta_o[...] = hn @ w1a_r[...] + b1_r[...]
            tb_o[...] = hn @ w1b_r[...]

    in_specs = [
        pl.BlockSpec((NB, D), lambda i: (i, 0)),
        pl.BlockSpec((NB, 4), lambda i: (i, 0)),
    ] + [pl.BlockSpec((NB, EW), lambda i: (i, 0))] * 8 + [
        _full((D, D)),
        _full((D, D)),
        _full((1, D)),
    ]
    out_specs = [
        pl.BlockSpec((NB, D), lambda i: (i, 0)),
        pl.BlockSpec((NB, 4), lambda i: (i, 0)),
    ]
    out_shape = [
        jax.ShapeDtypeStruct((NP, D), _f32),
        jax.ShapeDtypeStruct((NP, 4), _f32),
    ]
    if has_next:
        in_specs += [_full((D, D)), _full((D, D)), _full((1, D))]
        out_specs += [pl.BlockSpec((NB, D), lambda i: (i, 0)),
                      pl.BlockSpec((NB, D), lambda i: (i, 0))]
        out_shape += [jax.ShapeDtypeStruct((NP, D), _f32),
                      jax.ShapeDtypeStruct((NP, D), _f32)]
    return pl.pallas_call(
        body,
        grid=(NP // NB,),
        in_specs=in_specs,
        out_specs=out_specs,
        out_shape=out_shape,
    )


_node_call_mid = _make_node_call(True)
_node_call_last = _make_node_call(False)


def _pool_body(bid_r, h_r, out_o, acc, cnt):
    i = pl.program_id(0)

    @pl.when(i == 0)
    def _():
        acc[...] = jnp.zeros_like(acc)
        cnt[...] = jnp.zeros_like(cnt)

    ids = bid_r[0, 0, :]
    onehot = (ids[None, :] == lax.broadcasted_iota(_i32, (G, NB), 0))
    onehot = onehot.astype(_f32)
    acc[...] += onehot @ h_r[...]
    cnt[...] += jnp.broadcast_to(jnp.sum(onehot, axis=1)[:, None], (G, D))

    @pl.when(i == pl.num_programs(0) - 1)
    def _():
        out_o[...] = acc[...] / jnp.maximum(cnt[...], 1.0)


_pool_call = pl.pallas_call(
    _pool_body,
    grid=(NP // NB,),
    in_specs=[
        pl.BlockSpec((1, 1, NB), lambda i: (i, 0, 0)),
        pl.BlockSpec((NB, D), lambda i: (i, 0)),
    ],
    out_specs=pl.BlockSpec((G, D), lambda i: (0, 0)),
    out_shape=jax.ShapeDtypeStruct((G, D), _f32),
    scratch_shapes=[pltpu.VMEM((G, D), _f32), pltpu.VMEM((G, D), _f32)],
)


def _pack_bf16(t):
    t16 = t.astype(jnp.bfloat16).reshape(t.shape[0], PACKW, 2)
    return lax.bitcast_convert_type(t16, _i32)


def _unpack_bf16(t):
    t16 = lax.bitcast_convert_type(t, jnp.bfloat16)
    return t16.reshape(t.shape[0], D).astype(_f32)

@jax.jit
def kernel(x, pos, edge_index, batch_ids, W_in, b_in, msg_W1, msg_b1, msg_W2,
           msg_b2, pos_W, upd_W, upd_b, ln_g, ln_b):
    src = edge_index[0]
    dst = edge_index[1]
    xp = jnp.pad(x, ((0, NP - N), (0, 0)))
    pp = jnp.pad(pos, ((0, NP - N), (0, 1)))
    bid = jnp.pad(batch_ids, (0, NP - N), constant_values=G)
    bid = bid.reshape(NP // NB, 1, NB)
    zeros = jnp.zeros((NP, EW), _f32)

    h, ta, tb, p4 = _input_call(xp, pp, W_in, b_in[None], msg_W1[0, :D],
                                msg_W1[0, D:2 * D], msg_b1[0][None])
    for l in range(L):
        ew = (msg_W2[l], msg_b2[l][None], msg_W1[l, 2 * D][None],
              pos_W[l, :, 0][None], ln_g[l][None], ln_b[l][None])
        p4f = p4.reshape(-1)
        tap, tbp = _pack_bf16(ta), _pack_bf16(tb)
        ai0, bj0, sd0 = _gather_call(0)(tap, tbp, p4f, dst, src)
        ai1, bj1, sd1 = _gather_call(1)(tap, tbp, p4f, dst, src)
        e1_0, e2_0, e3_0 = _edge_call(_unpack_bf16(ai0), _unpack_bf16(bj0),
                                      sd0, *ew)
        sc0 = _scatter_call(0)(e1_0, e2_0, e3_0, dst, zeros)
        e1_1, e2_1, e3_1 = _edge_call(_unpack_bf16(ai1), _unpack_bf16(bj1),
                                      sd1, *ew)
        sc1 = _scatter_call(1)(e1_1, e2_1, e3_1, dst, zeros)
        aggs = (sc0[0], sc0[1], sc1[0], sc1[1], sc0[2], sc0[3], sc1[2], sc1[3])
        if l < L - 1:
            h, p4, ta, tb = _node_call_mid(
                h, p4, *aggs, upd_W[l, :D], upd_W[l, D:],
                upd_b[l][None], msg_W1[l + 1, :D], msg_W1[l + 1, D:2 * D],
                msg_b1[l + 1][None])
        else:
            h, p4 = _node_call_last(h, p4, *aggs, upd_W[l, :D], upd_W[l, D:],
                                    upd_b[l][None])

    gemb = _pool_call(bid, h)
    return (h[:N], gemb, p4[:N, 0:3])


# speedup vs baseline: 3.8960x; 3.8960x over previous
import functools

import jax
import jax.numpy as jnp
from jax import lax
from jax.experimental import pallas as pl
from jax.experimental.pallas import tpu as pltpu
from jax.experimental.pallas import tpu_sc as plsc

N = 10000
NP = 10240
E = 160000
D = 256
EW = 128
SDW = 16
G = 8
L = 3
CHUNK = 128
NCHUNK = E // CHUNK
NB = 1024
EB = 640
NTILE = 32
ROWS_PER_TILE = NP // 16

_f32 = jnp.float32
_i32 = jnp.int32


def _sc_mesh():
    return plsc.VectorSubcoreMesh(core_axis_name="c", subcore_axis_name="s",
                                  num_cores=2, num_subcores=16)


def _tile_range(s, total):
    per = total // 16
    rem = total - per * 16
    base = s * per + jnp.minimum(s, rem)
    nj = per + (s < rem).astype(_i32)
    return base, nj


def _gather_body(eoff, nchunks, ta, tb, p4, dst, src, outa, outb, outd,
                 idxg, idxo, bufs, ptab, bufd, sem0, sem1, semi0, semi1):
    c = lax.axis_index("c")
    s = lax.axis_index("s")
    base, nj = _tile_range(s, nchunks)

    def run(table, gidx, oidx, out, g_is_dst, dpred):
        @pl.when(dpred)
        def _():
            pltpu.sync_copy(p4, ptab)
        sems = (sem0, sem1)
        isems = (semi0, semi1)

        def idx_load_async(j, slot):
            off = (eoff + base + j) * CHUNK
            pltpu.async_copy(gidx.at[pl.ds(off, CHUNK)], idxg.at[slot],
                             isems[slot])

            @pl.when(dpred)
            def _():
                pltpu.async_copy(oidx.at[pl.ds(off, CHUNK)], idxo.at[slot],
                                 isems[slot])

        def idx_wait(j, slot):
            off = (eoff + base + j) * CHUNK
            pltpu.make_async_copy(gidx.at[pl.ds(off, CHUNK)], idxg.at[slot],
                                  isems[slot]).wait()

            @pl.when(dpred)
            def _():
                pltpu.make_async_copy(oidx.at[pl.ds(off, CHUNK)],
                                      idxo.at[slot], isems[slot]).wait()

        def fire(slot):
            pltpu.async_copy(table.at[idxg.at[slot]], bufs.at[slot], sems[slot])

        def finish(j, slot):
            off = (base + j) * CHUNK

            @pl.when(dpred)
            def _():
                def dc(i, carry2):
                    gv = idxg[slot, pl.ds(i * 16, 16)] * 4
                    ov = idxo[slot, pl.ds(i * 16, 16)] * 4
                    dv, sv = (gv, ov) if g_is_dst else (ov, gv)
                    dx = (plsc.load_gather(ptab, [dv])
                          - plsc.load_gather(ptab, [sv]))
                    dy = (plsc.load_gather(ptab, [dv + 1])
                          - plsc.load_gather(ptab, [sv + 1]))
                    dz = (plsc.load_gather(ptab, [dv + 2])
                          - plsc.load_gather(ptab, [sv + 2]))
                    dist2 = dx * dx + dy * dy + dz * dz
                    bufd[0, pl.ds(i * 16, 16)] = dx
                    bufd[1, pl.ds(i * 16, 16)] = dy
                    bufd[2, pl.ds(i * 16, 16)] = dz
                    bufd[3, pl.ds(i * 16, 16)] = dist2
                    return carry2

                lax.fori_loop(0, CHUNK // 16, dc, None)
                pltpu.sync_copy(bufd, outd.at[:, pl.ds(off, CHUNK)])

            pltpu.make_async_copy(table.at[idxg.at[slot]], bufs.at[slot],
                                  sems[slot]).wait()
            pltpu.sync_copy(bufs.at[slot], out.at[pl.ds(off, CHUNK), :])

        idx_load_async(0, 0)
        idx_wait(0, 0)
        fire(0)

        @pl.when(1 < nj)
        def _():
            idx_load_async(1, 1)

        def step(j, cur, nxt):
            @pl.when(j + 1 < nj)
            def _():
                idx_wait(j + 1, nxt)
                fire(nxt)
            finish(j, cur)

            @pl.when(j + 2 < nj)
            def _():
                idx_load_async(j + 2, cur)

        def lbody(j, carry):
            @pl.when(j % 2 == 0)
            def _():
                step(j, 0, 1)

            @pl.when(j % 2 == 1)
            def _():
                step(j, 1, 0)

            return carry

        lax.fori_loop(0, nj, lbody, None)

    @pl.when(c == 0)
    def _():
        run(ta, dst, src, outa, True, s < 8)

    @pl.when(c == 1)
    def _():
        run(tb, src, dst, outb, False, s >= 8)


PACKW = D // 2


@functools.cache
def _gather_call(half):
    e2 = E // 2
    return pl.kernel(
        functools.partial(_gather_body, half * (NCHUNK // 2), NCHUNK // 2),
        out_type=[jax.ShapeDtypeStruct((e2, PACKW), _i32),
                  jax.ShapeDtypeStruct((e2, PACKW), _i32),
                  jax.ShapeDtypeStruct((4, e2), _f32)],
        mesh=_sc_mesh(),
        compiler_params=pltpu.CompilerParams(needs_layout_passes=False),
        scratch_types=[
            pltpu.VMEM((2, CHUNK), _i32),
            pltpu.VMEM((2, CHUNK), _i32),
            pltpu.VMEM((2, CHUNK, PACKW), _i32),
            pltpu.VMEM((NP * 4,), _f32),
            pltpu.VMEM((4, CHUNK), _f32),
            pltpu.SemaphoreType.DMA,
            pltpu.SemaphoreType.DMA,
            pltpu.SemaphoreType.DMA,
            pltpu.SemaphoreType.DMA,
        ],
    )


def _zero_acc(s, zhbm, acc):
    rows = pl.ds(s * ROWS_PER_TILE, ROWS_PER_TILE)
    pltpu.sync_copy(zhbm.at[rows, :], acc.at[rows, :])


def _scatter_loop(eoff, e, dst, base, nj, idx2, buf2, acc, semi, semb):
    sems = (semi, semb)

    def load(j, slot):
        off = (base + j) * CHUNK
        goff = off + eoff * CHUNK
        pltpu.async_copy(dst.at[pl.ds(goff, CHUNK)], idx2.at[slot], sems[slot])
        pltpu.async_copy(e.at[pl.ds(off, CHUNK), :], buf2.at[slot], sems[slot])

    def finish(j, slot):
        off = (base + j) * CHUNK
        goff = off + eoff * CHUNK
        pltpu.make_async_copy(dst.at[pl.ds(goff, CHUNK)], idx2.at[slot],
                              sems[slot]).wait()
        pltpu.make_async_copy(e.at[pl.ds(off, CHUNK), :], buf2.at[slot],
                              sems[slot]).wait()
        pltpu.sync_copy(buf2.at[slot], acc.at[idx2.at[slot]], add=True)

    load(0, 0)

    def lbody(j, carry):
        @pl.when(j % 2 == 0)
        def _():
            @pl.when(j + 1 < nj)
            def _():
                load(j + 1, 1)
            finish(j, 0)

        @pl.when(j % 2 == 1)
        def _():
            @pl.when(j + 1 < nj)
            def _():
                load(j + 1, 0)
            finish(j, 1)

        return carry

    lax.fori_loop(0, nj, lbody, None)


def _scatter_body(eoff, nchunks, e1, e2, e3, dst, zhbm,
                  agg_a, agg_b, agg3a, agg3b, idx2, buf2, acc, semi, semb):
    c = lax.axis_index("c")
    s = lax.axis_index("s")
    rows = pl.ds(s * ROWS_PER_TILE, ROWS_PER_TILE)
    _zero_acc(s, zhbm, acc)
    plsc.subcore_barrier()

    base, nj = _tile_range(s, nchunks)

    @pl.when(c == 0)
    def _():
        _scatter_loop(eoff, e1, dst, base, nj, idx2, buf2, acc, semi, semb)

    @pl.when(c == 1)
    def _():
        _scatter_loop(eoff, e2, dst, base, nj, idx2, buf2, acc, semi, semb)

    plsc.subcore_barrier()

    @pl.when(c == 0)
    def _():
        pltpu.sync_copy(acc.at[rows, :], agg_a.at[rows, :])

    @pl.when(c == 1)
    def _():
        pltpu.sync_copy(acc.at[rows, :], agg_b.at[rows, :])

    _zero_acc(s, zhbm, acc)
    plsc.subcore_barrier()

    nc0 = nchunks // 2

    @pl.when(c == 0)
    def _():
        base2, nj2 = _tile_range(s, nc0)
        _scatter_loop(eoff, e3, dst, base2, nj2, idx2, buf2, acc, semi, semb)

    @pl.when(c == 1)
    def _():
        base2, nj2 = _tile_range(s, nchunks - nc0)
        _scatter_loop(eoff, e3, dst, nc0 + base2, nj2, idx2, buf2, acc,
                      semi, semb)

    plsc.subcore_barrier()

    @pl.when(c == 0)
    def _():
        pltpu.sync_copy(acc.at[rows, :], agg3a.at[rows, :])

    @pl.when(c == 1)
    def _():
        pltpu.sync_copy(acc.at[rows, :], agg3b.at[rows, :])


@functools.cache
def _scatter_call(half):
    return pl.kernel(
        functools.partial(_scatter_body, half * (NCHUNK // 2), NCHUNK // 2),
        out_type=[jax.ShapeDtypeStruct((NP, EW), _f32)] * 4,
        mesh=_sc_mesh(),
        scratch_types=[
            pltpu.VMEM((2, CHUNK), _i32),
            pltpu.VMEM((2, CHUNK, EW), _f32),
            pltpu.VMEM_SHARED((NP, EW), _f32),
            pltpu.SemaphoreType.DMA,
            pltpu.SemaphoreType.DMA,
        ],
    )


def _pack_cols(t):
    lo = lax.bitcast_convert_type(t[:, :PACKW].astype(jnp.bfloat16),
                                  jnp.int16).astype(_i32) & 0xFFFF
    hi = lax.bitcast_convert_type(t[:, PACKW:].astype(jnp.bfloat16),
                                  jnp.int16).astype(_i32) << 16
    return lo | hi


def _unpack_cols(x):
    lo = lax.bitcast_convert_type(x << 16, _f32)
    hi = lax.bitcast_convert_type(x & jnp.int32(-65536), _f32)
    return jnp.concatenate([lo, hi], axis=1)


def _input_body(x_r, p_r, wi_r, bi_r, w1a_r, w1b_r, b1_r, h_o, ta_o, tb_o, p4_o):
    h = x_r[...] @ wi_r[...] + bi_r[...]
    h_o[...] = h
    ta_o[...] = _pack_cols(h @ w1a_r[...] + b1_r[...])
    tb_o[...] = _pack_cols(h @ w1b_r[...])
    p4_o[...] = p_r[...]


def _full(shape):
    return pl.BlockSpec(shape, lambda i: (0, 0))


_input_call = pl.pallas_call(
    _input_body,
    grid=(NP // NB,),
    in_specs=[
        pl.BlockSpec((NB, D), lambda i: (i, 0)),
        pl.BlockSpec((NB, 4), lambda i: (i, 0)),
        _full((D, D)),
        _full((1, D)),
        _full((D, D)),
        _full((D, D)),
        _full((1, D)),
    ],
    out_specs=[
        pl.BlockSpec((NB, D), lambda i: (i, 0)),
        pl.BlockSpec((NB, PACKW), lambda i: (i, 0)),
        pl.BlockSpec((NB, PACKW), lambda i: (i, 0)),
        pl.BlockSpec((NB, 4), lambda i: (i, 0)),
    ],
    out_shape=[
        jax.ShapeDtypeStruct((NP, D), _f32),
        jax.ShapeDtypeStruct((NP, PACKW), _i32),
        jax.ShapeDtypeStruct((NP, PACKW), _i32),
        jax.ShapeDtypeStruct((NP, 4), _f32),
    ],
)


def _edge_body(ai_r, bj_r, sd_r, w2_r, b2_r, w1d_r, posw_r, g_r, bb_r,
               e1_o, e2_o, e3_o):
    sd = jnp.transpose(sd_r[...])
    dcol = sd[:, 0:3]
    dist2 = sd[:, 3:4]
    m = jnp.maximum(_unpack_cols(ai_r[...]) + _unpack_cols(bj_r[...])
                    + dist2 * w1d_r[...], 0.0)
    m = jnp.maximum(m @ w2_r[...] + b2_r[...], 0.0)
    mu = jnp.mean(m, axis=1, keepdims=True)
    var = jnp.mean(jnp.square(m - mu), axis=1, keepdims=True)
    m = (m - mu) * lax.rsqrt(var + 1e-5) * g_r[...] + bb_r[...]
    wgt = jnp.tanh(jnp.sum(m * posw_r[...], axis=1, keepdims=True))
    dw = dcol * wgt
    ones = jnp.ones((EB, 1), _f32)
    zpad = jnp.zeros((EB, EW - 4), _f32)
    e1_o[...] = m[:, 0:EW]
    e2_o[...] = m[:, EW:D]
    e3_o[...] = jnp.concatenate([dw, ones, zpad], axis=1)


E2 = E // 2

_edge_call = pl.pallas_call(
    _edge_body,
    grid=(E2 // EB,),
    in_specs=[
        pl.BlockSpec((EB, PACKW), lambda i: (i, 0)),
        pl.BlockSpec((EB, PACKW), lambda i: (i, 0)),
        pl.BlockSpec((4, EB), lambda i: (0, i)),
        _full((D, D)),
        _full((1, D)),
        _full((1, D)),
        _full((1, D)),
        _full((1, D)),
        _full((1, D)),
    ],
    out_specs=[
        pl.BlockSpec((EB, EW), lambda i: (i, 0)),
        pl.BlockSpec((EB, EW), lambda i: (i, 0)),
        pl.BlockSpec((EB, EW), lambda i: (i, 0)),
    ],
    out_shape=[
        jax.ShapeDtypeStruct((E2, EW), _f32),
        jax.ShapeDtypeStruct((E2, EW), _f32),
        jax.ShapeDtypeStruct((E2, EW), _f32),
    ],
)


def _make_node_call(has_next):
    def body(*refs):
        if has_next:
            (h_r, p_r, aa0_r, ab0_r, aa1_r, ab1_r, a3a0_r, a3b0_r, a3a1_r,
             a3b1_r, wua_r, wub_r, ub_r, w1a_r, w1b_r, b1_r,
             h_o, p_o, ta_o, tb_o) = refs
        else:
            (h_r, p_r, aa0_r, ab0_r, aa1_r, ab1_r, a3a0_r, a3b0_r, a3a1_r,
             a3b1_r, wua_r, wub_r, ub_r, h_o, p_o) = refs
        a3 = a3a0_r[...] + a3b0_r[...] + a3a1_r[...] + a3b1_r[...]
        deg = jnp.maximum(a3[:, 3:4], 1.0)
        magg = jnp.concatenate([aa0_r[...] + aa1_r[...],
                                ab0_r[...] + ab1_r[...]], axis=1) / deg
        hv = h_r[...]
        hu = jnp.maximum(hv @ wua_r[...] + magg @ wub_r[...] + ub_r[...], 0.0)
        hn = hv + hu
        p3 = p_r[...][:, 0:3] + a3[:, 0:3] / deg
        h_o[...] = hn
        p_o[...] = jnp.concatenate([p3, jnp.zeros((NB, 1), _f32)], axis=1)
        if has_next:
            ta_o[...] = _pack_cols(hn @ w1a_r[...] + b1_r[...])
            tb_o[...] = _pack_cols(hn @ w1b_r[...])

    in_specs = [
        pl.BlockSpec((NB, D), lambda i: (i, 0)),
        pl.BlockSpec((NB, 4), lambda i: (i, 0)),
    ] + [pl.BlockSpec((NB, EW), lambda i: (i, 0))] * 8 + [
        _full((D, D)),
        _full((D, D)),
        _full((1, D)),
    ]
    out_specs = [
        pl.BlockSpec((NB, D), lambda i: (i, 0)),
        pl.BlockSpec((NB, 4), lambda i: (i, 0)),
    ]
    out_shape = [
        jax.ShapeDtypeStruct((NP, D), _f32),
        jax.ShapeDtypeStruct((NP, 4), _f32),
    ]
    if has_next:
        in_specs += [_full((D, D)), _full((D, D)), _full((1, D))]
        out_specs += [pl.BlockSpec((NB, PACKW), lambda i: (i, 0)),
                      pl.BlockSpec((NB, PACKW), lambda i: (i, 0))]
        out_shape += [jax.ShapeDtypeStruct((NP, PACKW), _i32),
                      jax.ShapeDtypeStruct((NP, PACKW), _i32)]
    return pl.pallas_call(
        body,
        grid=(NP // NB,),
        in_specs=in_specs,
        out_specs=out_specs,
        out_shape=out_shape,
    )


_node_call_mid = _make_node_call(True)
_node_call_last = _make_node_call(False)


def _pool_body(bid_r, h_r, out_o, acc, cnt):
    i = pl.program_id(0)

    @pl.when(i == 0)
    def _():
        acc[...] = jnp.zeros_like(acc)
        cnt[...] = jnp.zeros_like(cnt)

    ids = bid_r[0, 0, :]
    onehot = (ids[None, :] == lax.broadcasted_iota(_i32, (G, NB), 0))
    onehot = onehot.astype(_f32)
    acc[...] += onehot @ h_r[...]
    cnt[...] += jnp.broadcast_to(jnp.sum(onehot, axis=1)[:, None], (G, D))

    @pl.when(i == pl.num_programs(0) - 1)
    def _():
        out_o[...] = acc[...] / jnp.maximum(cnt[...], 1.0)


_pool_call = pl.pallas_call(
    _pool_body,
    grid=(NP // NB,),
    in_specs=[
        pl.BlockSpec((1, 1, NB), lambda i: (i, 0, 0)),
        pl.BlockSpec((NB, D), lambda i: (i, 0)),
    ],
    out_specs=pl.BlockSpec((G, D), lambda i: (0, 0)),
    out_shape=jax.ShapeDtypeStruct((G, D), _f32),
    scratch_shapes=[pltpu.VMEM((G, D), _f32), pltpu.VMEM((G, D), _f32)],
)


@jax.jit
def kernel(x, pos, edge_index, batch_ids, W_in, b_in, msg_W1, msg_b1, msg_W2,
           msg_b2, pos_W, upd_W, upd_b, ln_g, ln_b):
    src = edge_index[0]
    dst = edge_index[1]
    xp = jnp.pad(x, ((0, NP - N), (0, 0)))
    pp = jnp.pad(pos, ((0, NP - N), (0, 1)))
    bid = jnp.pad(batch_ids, (0, NP - N), constant_values=G)
    bid = bid.reshape(NP // NB, 1, NB)
    zeros = jnp.zeros((NP, EW), _f32)

    h, ta, tb, p4 = _input_call(xp, pp, W_in, b_in[None], msg_W1[0, :D],
                                msg_W1[0, D:2 * D], msg_b1[0][None])
    for l in range(L):
        ew = (msg_W2[l], msg_b2[l][None], msg_W1[l, 2 * D][None],
              pos_W[l, :, 0][None], ln_g[l][None], ln_b[l][None])
        p4f = p4.reshape(-1)
        ai0, bj0, sd0 = _gather_call(0)(ta, tb, p4f, dst, src)
        ai1, bj1, sd1 = _gather_call(1)(ta, tb, p4f, dst, src)
        e1_0, e2_0, e3_0 = _edge_call(ai0, bj0, sd0, *ew)
        sc0 = _scatter_call(0)(e1_0, e2_0, e3_0, dst, zeros)
        e1_1, e2_1, e3_1 = _edge_call(ai1, bj1, sd1, *ew)
        sc1 = _scatter_call(1)(e1_1, e2_1, e3_1, dst, zeros)
        aggs = (sc0[0], sc0[1], sc1[0], sc1[1], sc0[2], sc0[3], sc1[2], sc1[3])
        if l < L - 1:
            h, p4, ta, tb = _node_call_mid(
                h, p4, *aggs, upd_W[l, :D], upd_W[l, D:],
                upd_b[l][None], msg_W1[l + 1, :D], msg_W1[l + 1, D:2 * D],
                msg_b1[l + 1][None])
        else:
            h, p4 = _node_call_last(h, p4, *aggs, upd_W[l, :D], upd_W[l, D:],
                                    upd_b[l][None])

    gemb = _pool_call(bid, h)
    return (h[:N], gemb, p4[:N, 0:3])
